# baseline (speedup 1.0000x reference)
import functools

import jax
import jax.numpy as jnp
from jax import lax
from jax.experimental import pallas as pl
from jax.experimental.pallas import tpu as pltpu
from jax.experimental.pallas import tpu_sc as plsc

TILE = 256
_NC = 2
_NS = 16
_NW = _NC * _NS


def _router_kernel(x_ref, wg_ref, bg_ref,
                   dest0_ref, dest1_ref, g0_ref, g1_ref,
                   tb_ref, *, tile, max_tiles):
    T = x_ref.shape[0]
    E = wg_ref.shape[1]
    logits = jnp.dot(x_ref[...], wg_ref[...],
                     preferred_element_type=jnp.float32) + bg_ref[...]
    col = jax.lax.broadcasted_iota(jnp.int32, logits.shape, 1)
    m1 = jnp.max(logits, axis=-1, keepdims=True)
    i1 = jnp.min(jnp.where(logits == m1, col, E), axis=-1, keepdims=True)
    rest = jnp.where(col == i1, -jnp.inf, logits)
    m2 = jnp.max(rest, axis=-1, keepdims=True)
    i2 = jnp.min(jnp.where(rest == m2, col, E), axis=-1, keepdims=True)
    e2v = jnp.exp(m2 - m1)
    s = 1.0 + e2v
    g0_ref[...] = jnp.broadcast_to(1.0 / s, g0_ref.shape)
    g1_ref[...] = jnp.broadcast_to(e2v / s, g1_ref.shape)

    onehot1 = (col == i1).astype(jnp.float32)
    onehot2 = (col == i2).astype(jnp.float32)
    ind = onehot1 + onehot2

    r = jax.lax.broadcasted_iota(jnp.int32, (T, T), 0)
    c = jax.lax.broadcasted_iota(jnp.int32, (T, T), 1)
    L = (r > c).astype(jnp.float32)
    rank = jnp.dot(L, ind, preferred_element_type=jnp.float32)

    counts = jnp.sum(ind, axis=0, keepdims=True)
    ntiles = jnp.floor((counts + (tile - 1)) / tile)
    ec = jax.lax.broadcasted_iota(jnp.int32, (E, E), 0)
    er = jax.lax.broadcasted_iota(jnp.int32, (E, E), 1)
    U = (ec < er).astype(jnp.float32)
    tcum = jnp.dot(ntiles, U, preferred_element_type=jnp.float32)
    offsets = tile * tcum

    base = rank + offsets
    dest0_ref[...] = jnp.sum(base * onehot1, axis=-1,
                             keepdims=True).astype(jnp.int32)
    dest1_ref[...] = jnp.sum(base * onehot2, axis=-1,
                             keepdims=True).astype(jnp.int32)

    pe = jax.lax.broadcasted_iota(jnp.int32, (E, 16), 0)
    pp = jax.lax.broadcasted_iota(jnp.int32, (E, 16), 1)
    M = (pe < pp).astype(jnp.float32)
    tb_ref[...] = jnp.dot(ntiles, M,
                          preferred_element_type=jnp.float32).astype(jnp.int32)


def _dispatch_body(chunk, x_hbm, d0_hbm, d1_hbm, xs_hbm,
                   rows_v, i0_v, i1_v, sem0, sem1):
    wid = lax.axis_index("s") * _NC + lax.axis_index("c")
    base = wid * chunk
    pltpu.sync_copy(d0_hbm.at[pl.ds(base, chunk)], i0_v)
    pltpu.sync_copy(d1_hbm.at[pl.ds(base, chunk)], i1_v)
    pltpu.sync_copy(x_hbm.at[pl.ds(base, chunk)], rows_v)
    cp0 = pltpu.async_copy(rows_v, xs_hbm.at[i0_v], sem0)
    cp1 = pltpu.async_copy(rows_v, xs_hbm.at[i1_v], sem1)
    cp0.wait()
    cp1.wait()


def _ffn_kernel(tb_ref, xs_hbm, w1_hbm, b1_ref, w2_hbm, b2_ref, ys_hbm,
                w1b, w2b, xst, yst, sems, *, tile, n_e):
    def w_fetch(e):
        p = e % 2
        c1 = pltpu.make_async_copy(w1_hbm.at[e], w1b.at[p], sems.at[2 * p])
        c2 = pltpu.make_async_copy(w2_hbm.at[e], w2b.at[p], sems.at[2 * p + 1])
        c1.start()
        c2.start()
        return c1, c2

    w_fetch(0)
    w_fetch(1)
    for e in range(n_e):
        p = e % 2
        pltpu.make_async_copy(w1_hbm.at[e], w1b.at[p], sems.at[2 * p]).wait()
        pltpu.make_async_copy(w2_hbm.at[e], w2b.at[p],
                              sems.at[2 * p + 1]).wait()

        def tile_body(t, carry):
            cin = pltpu.make_async_copy(
                xs_hbm.at[pl.ds(t * tile, tile)], xst, sems.at[4])
            cin.start()
            cin.wait()
            h = jnp.dot(xst[...], w1b[p],
                        preferred_element_type=jnp.float32) + b1_ref[e]
            h = 0.5 * h * (1.0 + jax.lax.erf(h * 0.7071067811865476))
            yst[...] = jnp.dot(h, w2b[p],
                               preferred_element_type=jnp.float32) + b2_ref[e]
            cout = pltpu.make_async_copy(
                yst, ys_hbm.at[pl.ds(t * tile, tile)], sems.at[5])
            cout.start()
            cout.wait()
            return carry

        lax.fori_loop(tb_ref[0, e], tb_ref[0, e + 1], tile_body, 0)
        if e + 2 < n_e:
            w_fetch(e + 2)


def _combine_body(chunk, H, ys_hbm, d0_hbm, d1_hbm, g0_hbm, g1_hbm, out_hbm,
                  rows0_v, rows1_v, i0_v, i1_v, g0_v, g1_v, sem0, sem1):
    wid = lax.axis_index("s") * _NC + lax.axis_index("c")
    base = wid * chunk
    pltpu.sync_copy(d0_hbm.at[pl.ds(base, chunk)], i0_v)
    pltpu.sync_copy(d1_hbm.at[pl.ds(base, chunk)], i1_v)
    pltpu.sync_copy(g0_hbm.at[pl.ds(base, chunk)], g0_v)
    pltpu.sync_copy(g1_hbm.at[pl.ds(base, chunk)], g1_v)
    cp0 = pltpu.async_copy(ys_hbm.at[i0_v], rows0_v, sem0)
    cp1 = pltpu.async_copy(ys_hbm.at[i1_v], rows1_v, sem1)
    cp0.wait()
    cp1.wait()

    def body(r, carry):
        a = g0_v[r]
        b = g1_v[r]
        for j in range(H // 16):
            sl = pl.ds(j * 16, 16)
            rows0_v[r, sl] = a * rows0_v[r, sl] + b * rows1_v[r, sl]
        return carry

    lax.fori_loop(0, chunk, body, 0)
    pltpu.sync_copy(rows0_v, out_hbm.at[pl.ds(base, chunk)])


def kernel(x, Wg, bg, W1, b1, W2, b2):
    B, S, H = x.shape
    T = B * S
    E, _, FF = W1.shape
    x_flat = x.reshape(T, H)
    max_tiles = 2 * T // TILE + E - 1
    NS_SLOTS = max_tiles * TILE
    chunk = T // _NW

    dest0, dest1, g0, g1, tb = pl.pallas_call(
        functools.partial(_router_kernel, tile=TILE, max_tiles=max_tiles),
        in_specs=[
            pl.BlockSpec((T, H), lambda: (0, 0)),
            pl.BlockSpec((H, E), lambda: (0, 0)),
            pl.BlockSpec((1, E), lambda: (0, 0)),
        ],
        out_specs=[
            pl.BlockSpec((T, 1), lambda: (0, 0)),
            pl.BlockSpec((T, 1), lambda: (0, 0)),
            pl.BlockSpec((T, 16), lambda: (0, 0)),
            pl.BlockSpec((T, 16), lambda: (0, 0)),
            pl.BlockSpec((1, 16), lambda: (0, 0)),
        ],
        out_shape=[
            jax.ShapeDtypeStruct((T, 1), jnp.int32),
            jax.ShapeDtypeStruct((T, 1), jnp.int32),
            jax.ShapeDtypeStruct((T, 16), jnp.float32),
            jax.ShapeDtypeStruct((T, 16), jnp.float32),
            jax.ShapeDtypeStruct((1, 16), jnp.int32),
        ],
    )(x_flat, Wg, bg.reshape(1, E))

    d0 = dest0.reshape(T)
    d1 = dest1.reshape(T)

    mesh = plsc.VectorSubcoreMesh(core_axis_name="c", subcore_axis_name="s")

    xs = pl.kernel(
        functools.partial(_dispatch_body, chunk),
        out_type=jax.ShapeDtypeStruct((NS_SLOTS, H), jnp.float32),
        mesh=mesh,
        scratch_types=[
            pltpu.VMEM((chunk, H), jnp.float32),
            pltpu.VMEM((chunk,), jnp.int32),
            pltpu.VMEM((chunk,), jnp.int32),
            pltpu.SemaphoreType.DMA,
            pltpu.SemaphoreType.DMA,
        ],
    )(x_flat, d0, d1)

    ys = pl.pallas_call(
        functools.partial(_ffn_kernel, tile=TILE, n_e=E),
        in_specs=[
            pl.BlockSpec(memory_space=pltpu.SMEM),
            pl.BlockSpec(memory_space=pl.ANY),
            pl.BlockSpec(memory_space=pl.ANY),
            pl.BlockSpec((E, 1, FF), lambda: (0, 0, 0)),
            pl.BlockSpec(memory_space=pl.ANY),
            pl.BlockSpec((E, 1, H), lambda: (0, 0, 0)),
        ],
        out_specs=pl.BlockSpec(memory_space=pl.ANY),
        out_shape=jax.ShapeDtypeStruct((NS_SLOTS, H), jnp.float32),
        scratch_shapes=[
            pltpu.VMEM((2, H, FF), jnp.float32),
            pltpu.VMEM((2, FF, H), jnp.float32),
            pltpu.VMEM((TILE, H), jnp.float32),
            pltpu.VMEM((TILE, H), jnp.float32),
            pltpu.SemaphoreType.DMA((6,)),
        ],
    )(tb, xs, W1, b1.reshape(E, 1, FF), W2, b2.reshape(E, 1, H))

    out = pl.kernel(
        functools.partial(_combine_body, chunk, H),
        out_type=jax.ShapeDtypeStruct((T, H), jnp.float32),
        mesh=mesh,
        scratch_types=[
            pltpu.VMEM((chunk, H), jnp.float32),
            pltpu.VMEM((chunk, H), jnp.float32),
            pltpu.VMEM((chunk,), jnp.int32),
            pltpu.VMEM((chunk,), jnp.int32),
            pltpu.VMEM((chunk, 16), jnp.float32),
            pltpu.VMEM((chunk, 16), jnp.float32),
            pltpu.SemaphoreType.DMA,
            pltpu.SemaphoreType.DMA,
        ],
    )(ys, d0, d1, g0, g1)

    return out.reshape(B, S, H)


# speedup vs baseline: 1.4544x; 1.4544x over previous
import functools

import jax
import jax.numpy as jnp
from jax import lax
from jax.experimental import pallas as pl
from jax.experimental.pallas import tpu as pltpu
from jax.experimental.pallas import tpu_sc as plsc

TILE = 256
_NC = 2
_NS = 16
_NW = _NC * _NS


def _router_kernel(x_ref, wg_ref, bg_ref,
                   dest0_ref, dest1_ref, g0_ref, g1_ref,
                   te_ref, act_ref, src_ref, kp_ref, eseq_ref, tbk_ref,
                   *, tile, max_tiles):
    T = x_ref.shape[0]
    E = wg_ref.shape[1]
    logits = jnp.dot(x_ref[...], wg_ref[...],
                     preferred_element_type=jnp.float32) + bg_ref[...]
    col = jax.lax.broadcasted_iota(jnp.int32, logits.shape, 1)
    m1 = jnp.max(logits, axis=-1, keepdims=True)
    i1 = jnp.min(jnp.where(logits == m1, col, E), axis=-1, keepdims=True)
    rest = jnp.where(col == i1, -jnp.inf, logits)
    m2 = jnp.max(rest, axis=-1, keepdims=True)
    i2 = jnp.min(jnp.where(rest == m2, col, E), axis=-1, keepdims=True)
    e2v = jnp.exp(m2 - m1)
    s = 1.0 + e2v
    g0_ref[...] = jnp.broadcast_to(1.0 / s, g0_ref.shape)
    g1_ref[...] = jnp.broadcast_to(e2v / s, g1_ref.shape)

    onehot1 = (col == i1).astype(jnp.float32)
    onehot2 = (col == i2).astype(jnp.float32)
    ind = onehot1 + onehot2

    r = jax.lax.broadcasted_iota(jnp.int32, (T, T), 0)
    c = jax.lax.broadcasted_iota(jnp.int32, (T, T), 1)
    L = (r > c).astype(jnp.float32)
    rank = jnp.dot(L, ind, preferred_element_type=jnp.float32)

    counts = jnp.sum(ind, axis=0, keepdims=True)
    ntiles = jnp.floor((counts + (tile - 1)) / tile)
    ec = jax.lax.broadcasted_iota(jnp.int32, (E, E), 0)
    er = jax.lax.broadcasted_iota(jnp.int32, (E, E), 1)
    U = (ec < er).astype(jnp.float32)
    tcum = jnp.dot(ntiles, U, preferred_element_type=jnp.float32)
    offsets = tile * tcum

    base = rank + offsets
    dest0_ref[...] = jnp.sum(base * onehot1, axis=-1,
                             keepdims=True).astype(jnp.int32)
    dest1_ref[...] = jnp.sum(base * onehot2, axis=-1,
                             keepdims=True).astype(jnp.int32)

    na = jnp.sum(ntiles)
    ti = jax.lax.broadcasted_iota(jnp.int32, (max_tiles, 1), 0).astype(
        jnp.float32)
    i_eff = jnp.minimum(ti, na - 1.0)
    tcum_b = jnp.broadcast_to(tcum, (max_tiles, E))
    te = jnp.sum((tcum_b <= i_eff).astype(jnp.float32), axis=-1,
                 keepdims=True) - 1.0
    te_ref[...] = te.astype(jnp.int32)
    act_ref[...] = (ti < na).astype(jnp.int32)
    src_ref[...] = i_eff.astype(jnp.int32)

    present = (ntiles > 0.5).astype(jnp.float32)
    prescum = jnp.dot(present, U, preferred_element_type=jnp.float32)
    m = jnp.sum(present)
    onehot_te = (jax.lax.broadcasted_iota(
        jnp.int32, (max_tiles, E), 1).astype(jnp.float32) == jnp.broadcast_to(
            te, (max_tiles, E))).astype(jnp.float32)
    kp_ref[...] = jnp.sum(onehot_te * prescum, axis=-1,
                          keepdims=True).astype(jnp.int32)
    I_E = (ec == er).astype(jnp.float32)
    dn = (((1,), (1,)), ((), ()))
    pres_c = jax.lax.dot_general(I_E, present, dn,
                                 preferred_element_type=jnp.float32)
    prescum_c = jax.lax.dot_general(I_E, prescum, dn,
                                    preferred_element_type=jnp.float32)
    tcum_c = jax.lax.dot_general(I_E, tcum, dn,
                                 preferred_element_type=jnp.float32)
    colk = jax.lax.broadcasted_iota(jnp.int32, (E, 16), 1).astype(jnp.float32)
    rowe = jax.lax.broadcasted_iota(jnp.int32, (E, 16), 0).astype(jnp.float32)
    sel = (pres_c > 0.5) & (jnp.broadcast_to(prescum_c, (E, 16)) == colk)
    self_f = sel.astype(jnp.float32)
    eseq_ref[...] = jnp.sum(self_f * rowe, axis=0,
                            keepdims=True).astype(jnp.int32)
    colk_row = jax.lax.broadcasted_iota(jnp.int32, (1, 16), 1).astype(
        jnp.float32)
    tbk_ref[...] = (jnp.sum(self_f * jnp.broadcast_to(tcum_c, (E, 16)),
                            axis=0, keepdims=True)
                    + na * (colk_row >= m)).astype(jnp.int32)


def _dispatch_body(chunk, x_hbm, d0_hbm, d1_hbm, xs_hbm,
                   rows_v, i0_v, i1_v, sem0, sem1):
    wid = lax.axis_index("s") * _NC + lax.axis_index("c")
    base = wid * chunk
    pltpu.sync_copy(d0_hbm.at[pl.ds(base, chunk)], i0_v)
    pltpu.sync_copy(d1_hbm.at[pl.ds(base, chunk)], i1_v)
    pltpu.sync_copy(x_hbm.at[pl.ds(base, chunk)], rows_v)
    cp0 = pltpu.async_copy(rows_v, xs_hbm.at[i0_v], sem0)
    cp1 = pltpu.async_copy(rows_v, xs_hbm.at[i1_v], sem1)
    cp0.wait()
    cp1.wait()


def _ffn_kernel(te_ref, act_ref, src_ref, kp_ref, eseq_ref, tbk_ref,
                xs_ref, b1_ref, b2_ref, w1_hbm, w2_hbm, ys_ref,
                w1r, w2r, s1, s2, *, ffh, n_e):
    i = pl.program_id(0)
    k = kp_ref[i]

    def w_copies(kk):
        e = eseq_ref[kk]
        a = (kk % 2) * 2
        return (
            pltpu.make_async_copy(w1_hbm.at[e, :, pl.ds(0, ffh)],
                                  w1r.at[a], s1.at[a]),
            pltpu.make_async_copy(w1_hbm.at[e, :, pl.ds(ffh, ffh)],
                                  w1r.at[a + 1], s1.at[a + 1]),
            pltpu.make_async_copy(w2_hbm.at[e, pl.ds(0, ffh), :],
                                  w2r.at[a], s2.at[a]),
            pltpu.make_async_copy(w2_hbm.at[e, pl.ds(ffh, ffh), :],
                                  w2r.at[a + 1], s2.at[a + 1]),
        )

    def issue(kk):
        for c in w_copies(kk):
            c.start()

    def wait(kk):
        for c in w_copies(kk):
            c.wait()

    active = act_ref[i] == 1
    first = active & (i == tbk_ref[k])

    @pl.when(first & (i == 0))
    def _():
        issue(0)

        @pl.when(tbk_ref[2] > tbk_ref[1])
        def _():
            issue(1)

    @pl.when(first)
    def _():
        @pl.when((i > 0) & (tbk_ref[k + 2] > tbk_ref[k + 1]))
        def _():
            issue(k + 1)
        wait(k)

    @pl.when(active)
    def _():
        a = (k % 2) * 2
        b1v = b1_ref[0]
        xb = xs_ref[...]
        h0 = jnp.dot(xb, w1r[a],
                     preferred_element_type=jnp.float32) + b1v[:, :ffh]
        h0 = 0.5 * h0 * (1.0 + jax.lax.erf(h0 * 0.7071067811865476))
        h1 = jnp.dot(xb, w1r[a + 1],
                     preferred_element_type=jnp.float32) + b1v[:, ffh:]
        h1 = 0.5 * h1 * (1.0 + jax.lax.erf(h1 * 0.7071067811865476))
        y = jnp.dot(h0, w2r[a], preferred_element_type=jnp.float32)
        y = y + jnp.dot(h1, w2r[a + 1], preferred_element_type=jnp.float32)
        ys_ref[...] = y + b2_ref[0]


def _combine_body(chunk, H, ys_hbm, d0_hbm, d1_hbm, g0_hbm, g1_hbm, out_hbm,
                  rows0_v, rows1_v, i0_v, i1_v, g0_v, g1_v, sem0, sem1):
    wid = lax.axis_index("s") * _NC + lax.axis_index("c")
    base = wid * chunk
    pltpu.sync_copy(d0_hbm.at[pl.ds(base, chunk)], i0_v)
    pltpu.sync_copy(d1_hbm.at[pl.ds(base, chunk)], i1_v)
    pltpu.sync_copy(g0_hbm.at[pl.ds(base, chunk)], g0_v)
    pltpu.sync_copy(g1_hbm.at[pl.ds(base, chunk)], g1_v)
    cp0 = pltpu.async_copy(ys_hbm.at[i0_v], rows0_v, sem0)
    cp1 = pltpu.async_copy(ys_hbm.at[i1_v], rows1_v, sem1)
    cp0.wait()
    cp1.wait()

    def body(r, carry):
        a = g0_v[r]
        b = g1_v[r]
        for j in range(H // 16):
            sl = pl.ds(j * 16, 16)
            rows0_v[r, sl] = a * rows0_v[r, sl] + b * rows1_v[r, sl]
        return carry

    lax.fori_loop(0, chunk, body, 0)
    pltpu.sync_copy(rows0_v, out_hbm.at[pl.ds(base, chunk)])


def kernel(x, Wg, bg, W1, b1, W2, b2):
    B, S, H = x.shape
    T = B * S
    E, _, FF = W1.shape
    x_flat = x.reshape(T, H)
    max_tiles = 2 * T // TILE + E - 1
    NS_SLOTS = max_tiles * TILE
    chunk = T // _NW

    dest0, dest1, g0, g1, te, act, src, kp, eseq, tbk = pl.pallas_call(
        functools.partial(_router_kernel, tile=TILE, max_tiles=max_tiles),
        in_specs=[
            pl.BlockSpec((T, H), lambda: (0, 0)),
            pl.BlockSpec((H, E), lambda: (0, 0)),
            pl.BlockSpec((1, E), lambda: (0, 0)),
        ],
        out_specs=[
            pl.BlockSpec((T, 1), lambda: (0, 0)),
            pl.BlockSpec((T, 1), lambda: (0, 0)),
            pl.BlockSpec((T, 16), lambda: (0, 0)),
            pl.BlockSpec((T, 16), lambda: (0, 0)),
            pl.BlockSpec((max_tiles, 1), lambda: (0, 0)),
            pl.BlockSpec((max_tiles, 1), lambda: (0, 0)),
            pl.BlockSpec((max_tiles, 1), lambda: (0, 0)),
            pl.BlockSpec((max_tiles, 1), lambda: (0, 0)),
            pl.BlockSpec((1, 16), lambda: (0, 0)),
            pl.BlockSpec((1, 16), lambda: (0, 0)),
        ],
        out_shape=[
            jax.ShapeDtypeStruct((T, 1), jnp.int32),
            jax.ShapeDtypeStruct((T, 1), jnp.int32),
            jax.ShapeDtypeStruct((T, 16), jnp.float32),
            jax.ShapeDtypeStruct((T, 16), jnp.float32),
            jax.ShapeDtypeStruct((max_tiles, 1), jnp.int32),
            jax.ShapeDtypeStruct((max_tiles, 1), jnp.int32),
            jax.ShapeDtypeStruct((max_tiles, 1), jnp.int32),
            jax.ShapeDtypeStruct((max_tiles, 1), jnp.int32),
            jax.ShapeDtypeStruct((1, 16), jnp.int32),
            jax.ShapeDtypeStruct((1, 16), jnp.int32),
        ],
    )(x_flat, Wg, bg.reshape(1, E))

    d0 = dest0.reshape(T)
    d1 = dest1.reshape(T)

    mesh = plsc.VectorSubcoreMesh(core_axis_name="c", subcore_axis_name="s")

    xs = pl.kernel(
        functools.partial(_dispatch_body, chunk),
        out_type=jax.ShapeDtypeStruct((NS_SLOTS, H), jnp.float32),
        mesh=mesh,
        scratch_types=[
            pltpu.VMEM((chunk, H), jnp.float32),
            pltpu.VMEM((chunk,), jnp.int32),
            pltpu.VMEM((chunk,), jnp.int32),
            pltpu.SemaphoreType.DMA,
            pltpu.SemaphoreType.DMA,
        ],
    )(x_flat, d0, d1)

    ys = pl.pallas_call(
        functools.partial(_ffn_kernel, ffh=FF // 2, n_e=E),
        grid_spec=pltpu.PrefetchScalarGridSpec(
            num_scalar_prefetch=6,
            grid=(max_tiles,),
            in_specs=[
                pl.BlockSpec((TILE, H),
                             lambda i, te, a, sr, kp, es, tb: (sr[i], 0)),
                pl.BlockSpec((1, 1, FF),
                             lambda i, te, a, sr, kp, es, tb: (te[i], 0, 0)),
                pl.BlockSpec((1, 1, H),
                             lambda i, te, a, sr, kp, es, tb: (te[i], 0, 0)),
                pl.BlockSpec(memory_space=pl.ANY),
                pl.BlockSpec(memory_space=pl.ANY),
            ],
            out_specs=pl.BlockSpec((TILE, H),
                                   lambda i, te, a, sr, kp, es, tb: (sr[i], 0)),
            scratch_shapes=[
                pltpu.VMEM((4, H, FF // 2), jnp.float32),
                pltpu.VMEM((4, FF // 2, H), jnp.float32),
                pltpu.SemaphoreType.DMA((4,)),
                pltpu.SemaphoreType.DMA((4,)),
            ],
        ),
        out_shape=jax.ShapeDtypeStruct((NS_SLOTS, H), jnp.float32),
    )(te.reshape(-1), act.reshape(-1), src.reshape(-1), kp.reshape(-1),
      eseq.reshape(-1), tbk.reshape(-1),
      xs, b1.reshape(E, 1, FF), b2.reshape(E, 1, H), W1, W2)

    out = pl.kernel(
        functools.partial(_combine_body, chunk, H),
        out_type=jax.ShapeDtypeStruct((T, H), jnp.float32),
        mesh=mesh,
        scratch_types=[
            pltpu.VMEM((chunk, H), jnp.float32),
            pltpu.VMEM((chunk, H), jnp.float32),
            pltpu.VMEM((chunk,), jnp.int32),
            pltpu.VMEM((chunk,), jnp.int32),
            pltpu.VMEM((chunk, 16), jnp.float32),
            pltpu.VMEM((chunk, 16), jnp.float32),
            pltpu.SemaphoreType.DMA,
            pltpu.SemaphoreType.DMA,
        ],
    )(ys, d0, d1, g0, g1)

    return out.reshape(B, S, H)


# speedup vs baseline: 1.4778x; 1.0161x over previous
import functools

import jax
import jax.numpy as jnp
from jax import lax
from jax.experimental import pallas as pl
from jax.experimental.pallas import tpu as pltpu
from jax.experimental.pallas import tpu_sc as plsc

TILE = 256
_NC = 2
_NS = 16
_NW = _NC * _NS


def _router_kernel(x_ref, wg_ref, bg_ref,
                   dest0_ref, dest1_ref, g0_ref, g1_ref,
                   te_ref, act_ref, src_ref, kp_ref, eseq_ref, tbk_ref,
                   *, tile, max_tiles):
    T = x_ref.shape[0]
    E = wg_ref.shape[1]
    logits = jnp.dot(x_ref[...], wg_ref[...],
                     preferred_element_type=jnp.float32) + bg_ref[...]
    col = jax.lax.broadcasted_iota(jnp.int32, logits.shape, 1)
    m1 = jnp.max(logits, axis=-1, keepdims=True)
    i1 = jnp.min(jnp.where(logits == m1, col, E), axis=-1, keepdims=True)
    rest = jnp.where(col == i1, -jnp.inf, logits)
    m2 = jnp.max(rest, axis=-1, keepdims=True)
    i2 = jnp.min(jnp.where(rest == m2, col, E), axis=-1, keepdims=True)
    e2v = jnp.exp(m2 - m1)
    s = 1.0 + e2v
    g0_ref[...] = jnp.broadcast_to(1.0 / s, g0_ref.shape)
    g1_ref[...] = jnp.broadcast_to(e2v / s, g1_ref.shape)

    onehot1 = (col == i1).astype(jnp.float32)
    onehot2 = (col == i2).astype(jnp.float32)
    ind = onehot1 + onehot2

    r = jax.lax.broadcasted_iota(jnp.int32, (T, T), 0)
    c = jax.lax.broadcasted_iota(jnp.int32, (T, T), 1)
    L = (r > c).astype(jnp.float32)
    rank = jnp.dot(L, ind, preferred_element_type=jnp.float32)

    counts = jnp.sum(ind, axis=0, keepdims=True)
    ntiles = jnp.floor((counts + (tile - 1)) / tile)
    ec = jax.lax.broadcasted_iota(jnp.int32, (E, E), 0)
    er = jax.lax.broadcasted_iota(jnp.int32, (E, E), 1)
    U = (ec < er).astype(jnp.float32)
    tcum = jnp.dot(ntiles, U, preferred_element_type=jnp.float32)
    offsets = tile * tcum

    base = rank + offsets
    dest0_ref[...] = jnp.sum(base * onehot1, axis=-1,
                             keepdims=True).astype(jnp.int32)
    dest1_ref[...] = jnp.sum(base * onehot2, axis=-1,
                             keepdims=True).astype(jnp.int32)

    na = jnp.sum(ntiles)
    ti = jax.lax.broadcasted_iota(jnp.int32, (max_tiles, 1), 0).astype(
        jnp.float32)
    i_eff = jnp.minimum(ti, na - 1.0)
    tcum_b = jnp.broadcast_to(tcum, (max_tiles, E))
    te = jnp.sum((tcum_b <= i_eff).astype(jnp.float32), axis=-1,
                 keepdims=True) - 1.0
    te_ref[...] = te.astype(jnp.int32)
    act_ref[...] = (ti < na).astype(jnp.int32)
    src_ref[...] = i_eff.astype(jnp.int32)

    present = (ntiles > 0.5).astype(jnp.float32)
    prescum = jnp.dot(present, U, preferred_element_type=jnp.float32)
    m = jnp.sum(present)
    onehot_te = (jax.lax.broadcasted_iota(
        jnp.int32, (max_tiles, E), 1).astype(jnp.float32) == jnp.broadcast_to(
            te, (max_tiles, E))).astype(jnp.float32)
    kp_ref[...] = jnp.sum(onehot_te * prescum, axis=-1,
                          keepdims=True).astype(jnp.int32)
    I_E = (ec == er).astype(jnp.float32)
    dn = (((1,), (1,)), ((), ()))
    pres_c = jax.lax.dot_general(I_E, present, dn,
                                 preferred_element_type=jnp.float32)
    prescum_c = jax.lax.dot_general(I_E, prescum, dn,
                                    preferred_element_type=jnp.float32)
    tcum_c = jax.lax.dot_general(I_E, tcum, dn,
                                 preferred_element_type=jnp.float32)
    colk = jax.lax.broadcasted_iota(jnp.int32, (E, 16), 1).astype(jnp.float32)
    rowe = jax.lax.broadcasted_iota(jnp.int32, (E, 16), 0).astype(jnp.float32)
    sel = (pres_c > 0.5) & (jnp.broadcast_to(prescum_c, (E, 16)) == colk)
    self_f = sel.astype(jnp.float32)
    eseq_ref[...] = jnp.sum(self_f * rowe, axis=0,
                            keepdims=True).astype(jnp.int32)
    colk_row = jax.lax.broadcasted_iota(jnp.int32, (1, 16), 1).astype(
        jnp.float32)
    tbk_ref[...] = (jnp.sum(self_f * jnp.broadcast_to(tcum_c, (E, 16)),
                            axis=0, keepdims=True)
                    + na * (colk_row >= m)).astype(jnp.int32)


def _dispatch_body(chunk, x_hbm, d0_hbm, d1_hbm, xs_hbm,
                   rows_v, i0_v, i1_v, sem0, sem1, sem2):
    wid = lax.axis_index("s") * _NC + lax.axis_index("c")
    base = wid * chunk
    ld0 = pltpu.async_copy(d0_hbm.at[pl.ds(base, chunk)], i0_v, sem0)
    ld1 = pltpu.async_copy(d1_hbm.at[pl.ds(base, chunk)], i1_v, sem1)
    ldx = pltpu.async_copy(x_hbm.at[pl.ds(base, chunk)], rows_v, sem2)
    ld0.wait()
    ld1.wait()
    ldx.wait()
    cp0 = pltpu.async_copy(rows_v, xs_hbm.at[i0_v], sem0)
    cp1 = pltpu.async_copy(rows_v, xs_hbm.at[i1_v], sem1)
    cp0.wait()
    cp1.wait()


def _ffn_kernel(te_ref, act_ref, src_ref, kp_ref, eseq_ref, tbk_ref,
                xs_ref, b1_ref, b2_ref, w1_hbm, w2_hbm, ys_ref,
                w1r, w2r, s1, s2, *, ffh, n_e):
    i = pl.program_id(0)
    k = kp_ref[i]

    def w_copies(kk):
        e = eseq_ref[kk]
        a = (kk % 2) * 2
        return (
            pltpu.make_async_copy(w1_hbm.at[e, :, pl.ds(0, ffh)],
                                  w1r.at[a], s1.at[a]),
            pltpu.make_async_copy(w1_hbm.at[e, :, pl.ds(ffh, ffh)],
                                  w1r.at[a + 1], s1.at[a + 1]),
            pltpu.make_async_copy(w2_hbm.at[e, pl.ds(0, ffh), :],
                                  w2r.at[a], s2.at[a]),
            pltpu.make_async_copy(w2_hbm.at[e, pl.ds(ffh, ffh), :],
                                  w2r.at[a + 1], s2.at[a + 1]),
        )

    def issue(kk):
        for c in w_copies(kk):
            c.start()

    def wait(kk):
        for c in w_copies(kk):
            c.wait()

    active = act_ref[i] == 1
    first = active & (i == tbk_ref[k])

    @pl.when(first & (i == 0))
    def _():
        issue(0)

        @pl.when(tbk_ref[2] > tbk_ref[1])
        def _():
            issue(1)

    @pl.when(first)
    def _():
        @pl.when((i > 0) & (tbk_ref[k + 2] > tbk_ref[k + 1]))
        def _():
            issue(k + 1)
        wait(k)

    @pl.when(active)
    def _():
        a = (k % 2) * 2
        b1v = b1_ref[0]
        xb = xs_ref[...]
        h0 = jnp.dot(xb, w1r[a],
                     preferred_element_type=jnp.float32) + b1v[:, :ffh]
        h0 = 0.5 * h0 * (1.0 + jax.lax.erf(h0 * 0.7071067811865476))
        h1 = jnp.dot(xb, w1r[a + 1],
                     preferred_element_type=jnp.float32) + b1v[:, ffh:]
        h1 = 0.5 * h1 * (1.0 + jax.lax.erf(h1 * 0.7071067811865476))
        y = jnp.dot(h0, w2r[a], preferred_element_type=jnp.float32)
        y = y + jnp.dot(h1, w2r[a + 1], preferred_element_type=jnp.float32)
        ys_ref[...] = y + b2_ref[0]


def _combine_body(chunk, H, ys_hbm, d0_hbm, d1_hbm, g0_hbm, g1_hbm, out_hbm,
                  rows0_v, rows1_v, i0_v, i1_v, g0_v, g1_v, sem0, sem1):
    wid = lax.axis_index("s") * _NC + lax.axis_index("c")
    base = wid * chunk
    pltpu.sync_copy(d0_hbm.at[pl.ds(base, chunk)], i0_v)
    pltpu.sync_copy(d1_hbm.at[pl.ds(base, chunk)], i1_v)
    pltpu.sync_copy(g0_hbm.at[pl.ds(base, chunk)], g0_v)
    pltpu.sync_copy(g1_hbm.at[pl.ds(base, chunk)], g1_v)

    sub = chunk // 4

    def gathers(j):
        sl = pl.ds(j * sub, sub)
        return (pltpu.async_copy(ys_hbm.at[i0_v.at[sl]], rows0_v.at[sl], sem0),
                pltpu.async_copy(ys_hbm.at[i1_v.at[sl]], rows1_v.at[sl], sem1))

    pend = gathers(0)
    for j in range(4):
        cur = pend
        if j < 3:
            pend = gathers(j + 1)
        cur[0].wait()
        cur[1].wait()

        def body(r, carry):
            a = g0_v[r]
            b = g1_v[r]
            for jj in range(H // 16):
                sl = pl.ds(jj * 16, 16)
                rows0_v[r, sl] = a * rows0_v[r, sl] + b * rows1_v[r, sl]
            return carry

        lax.fori_loop(j * sub, (j + 1) * sub, body, 0)
    pltpu.sync_copy(rows0_v, out_hbm.at[pl.ds(base, chunk)])


def kernel(x, Wg, bg, W1, b1, W2, b2):
    B, S, H = x.shape
    T = B * S
    E, _, FF = W1.shape
    x_flat = x.reshape(T, H)
    max_tiles = 2 * T // TILE + E - 1
    NS_SLOTS = max_tiles * TILE
    chunk = T // _NW

    dest0, dest1, g0, g1, te, act, src, kp, eseq, tbk = pl.pallas_call(
        functools.partial(_router_kernel, tile=TILE, max_tiles=max_tiles),
        in_specs=[
            pl.BlockSpec((T, H), lambda: (0, 0)),
            pl.BlockSpec((H, E), lambda: (0, 0)),
            pl.BlockSpec((1, E), lambda: (0, 0)),
        ],
        out_specs=[
            pl.BlockSpec((T, 1), lambda: (0, 0)),
            pl.BlockSpec((T, 1), lambda: (0, 0)),
            pl.BlockSpec((T, 16), lambda: (0, 0)),
            pl.BlockSpec((T, 16), lambda: (0, 0)),
            pl.BlockSpec((max_tiles, 1), lambda: (0, 0)),
            pl.BlockSpec((max_tiles, 1), lambda: (0, 0)),
            pl.BlockSpec((max_tiles, 1), lambda: (0, 0)),
            pl.BlockSpec((max_tiles, 1), lambda: (0, 0)),
            pl.BlockSpec((1, 16), lambda: (0, 0)),
            pl.BlockSpec((1, 16), lambda: (0, 0)),
        ],
        out_shape=[
            jax.ShapeDtypeStruct((T, 1), jnp.int32),
            jax.ShapeDtypeStruct((T, 1), jnp.int32),
            jax.ShapeDtypeStruct((T, 16), jnp.float32),
            jax.ShapeDtypeStruct((T, 16), jnp.float32),
            jax.ShapeDtypeStruct((max_tiles, 1), jnp.int32),
            jax.ShapeDtypeStruct((max_tiles, 1), jnp.int32),
            jax.ShapeDtypeStruct((max_tiles, 1), jnp.int32),
            jax.ShapeDtypeStruct((max_tiles, 1), jnp.int32),
            jax.ShapeDtypeStruct((1, 16), jnp.int32),
            jax.ShapeDtypeStruct((1, 16), jnp.int32),
        ],
    )(x_flat, Wg, bg.reshape(1, E))

    d0 = dest0.reshape(T)
    d1 = dest1.reshape(T)

    mesh = plsc.VectorSubcoreMesh(core_axis_name="c", subcore_axis_name="s")

    xs = pl.kernel(
        functools.partial(_dispatch_body, chunk),
        out_type=jax.ShapeDtypeStruct((NS_SLOTS, H), jnp.float32),
        mesh=mesh,
        scratch_types=[
            pltpu.VMEM((chunk, H), jnp.float32),
            pltpu.VMEM((chunk,), jnp.int32),
            pltpu.VMEM((chunk,), jnp.int32),
            pltpu.SemaphoreType.DMA,
            pltpu.SemaphoreType.DMA,
            pltpu.SemaphoreType.DMA,
        ],
    )(x_flat, d0, d1)

    ys = pl.pallas_call(
        functools.partial(_ffn_kernel, ffh=FF // 2, n_e=E),
        grid_spec=pltpu.PrefetchScalarGridSpec(
            num_scalar_prefetch=6,
            grid=(max_tiles,),
            in_specs=[
                pl.BlockSpec((TILE, H),
                             lambda i, te, a, sr, kp, es, tb: (sr[i], 0)),
                pl.BlockSpec((1, 1, FF),
                             lambda i, te, a, sr, kp, es, tb: (te[i], 0, 0)),
                pl.BlockSpec((1, 1, H),
                             lambda i, te, a, sr, kp, es, tb: (te[i], 0, 0)),
                pl.BlockSpec(memory_space=pl.ANY),
                pl.BlockSpec(memory_space=pl.ANY),
            ],
            out_specs=pl.BlockSpec((TILE, H),
                                   lambda i, te, a, sr, kp, es, tb: (sr[i], 0)),
            scratch_shapes=[
                pltpu.VMEM((4, H, FF // 2), jnp.float32),
                pltpu.VMEM((4, FF // 2, H), jnp.float32),
                pltpu.SemaphoreType.DMA((4,)),
                pltpu.SemaphoreType.DMA((4,)),
            ],
        ),
        out_shape=jax.ShapeDtypeStruct((NS_SLOTS, H), jnp.float32),
    )(te.reshape(-1), act.reshape(-1), src.reshape(-1), kp.reshape(-1),
      eseq.reshape(-1), tbk.reshape(-1),
      xs, b1.reshape(E, 1, FF), b2.reshape(E, 1, H), W1, W2)

    out = pl.kernel(
        functools.partial(_combine_body, chunk, H),
        out_type=jax.ShapeDtypeStruct((T, H), jnp.float32),
        mesh=mesh,
        scratch_types=[
            pltpu.VMEM((chunk, H), jnp.float32),
            pltpu.VMEM((chunk, H), jnp.float32),
            pltpu.VMEM((chunk,), jnp.int32),
            pltpu.VMEM((chunk,), jnp.int32),
            pltpu.VMEM((chunk, 16), jnp.float32),
            pltpu.VMEM((chunk, 16), jnp.float32),
            pltpu.SemaphoreType.DMA,
            pltpu.SemaphoreType.DMA,
        ],
    )(ys, d0, d1, g0, g1)

    return out.reshape(B, S, H)
